# Initial kernel scaffold; baseline (speedup 1.0000x reference)
import jax
import jax.numpy as jnp
from jax.experimental import pallas as pl


def kernel(hidden_states, W):
    raise NotImplementedError("write your pallas kernel here")


# speedup vs baseline: 1.1347x; 1.1347x over previous
import jax
import jax.numpy as jnp
from jax.experimental import pallas as pl

TOP_K = 8
NUM_EXPERTS = 64
HIDDEN = 2048
BLOCK = 512
_IDX_MASK = NUM_EXPERTS - 1


def _router_body(hs_ref, wt_ref, probs_ref, scores_ref, idx_ref):
    x = hs_ref[...]
    logits = jax.lax.dot_general(
        x, wt_ref[...], (((1,), (0,)), ((), ())),
        preferred_element_type=jnp.float32,
    )
    m = jnp.max(logits, axis=-1, keepdims=True)
    e = jnp.exp(logits - m)
    s = jnp.sum(e, axis=-1, keepdims=True)
    p = e / s
    probs_ref[...] = p

    bits = jax.lax.bitcast_convert_type(p, jnp.int32)
    rev_iota = _IDX_MASK - jax.lax.broadcasted_iota(jnp.int32, p.shape, 1)
    key = (bits & ~_IDX_MASK) | rev_iota
    picks = []
    for _ in range(TOP_K):
        mk = jnp.max(key, axis=-1, keepdims=True)
        picks.append(mk)
        key = jnp.where(key == mk, jnp.iinfo(jnp.int32).min, key)
    k8 = jnp.concatenate(picks, axis=-1)
    idx = _IDX_MASK - (k8 & _IDX_MASK)
    v = jax.lax.bitcast_convert_type(k8 & ~_IDX_MASK, jnp.float32)
    scores_ref[...] = v / jnp.sum(v, axis=-1, keepdims=True)
    idx_ref[...] = idx


@jax.jit
def kernel(hidden_states, W):
    hs = hidden_states.reshape(-1, HIDDEN)
    n = hs.shape[0]
    wt = W.T
    grid = (n // BLOCK,)
    probs, scores, idx = pl.pallas_call(
        _router_body,
        grid=grid,
        in_specs=[
            pl.BlockSpec((BLOCK, HIDDEN), lambda i: (i, 0)),
            pl.BlockSpec((HIDDEN, NUM_EXPERTS), lambda i: (0, 0)),
        ],
        out_specs=[
            pl.BlockSpec((BLOCK, NUM_EXPERTS), lambda i: (i, 0)),
            pl.BlockSpec((BLOCK, TOP_K), lambda i: (i, 0)),
            pl.BlockSpec((BLOCK, TOP_K), lambda i: (i, 0)),
        ],
        out_shape=[
            jax.ShapeDtypeStruct((n, NUM_EXPERTS), jnp.float32),
            jax.ShapeDtypeStruct((n, TOP_K), jnp.float32),
            jax.ShapeDtypeStruct((n, TOP_K), jnp.int32),
        ],
    )(hs, wt)
    return (probs, scores, idx)


# speedup vs baseline: 1.2946x; 1.1409x over previous
import jax
import jax.numpy as jnp
from jax.experimental import pallas as pl

TOP_K = 8
NUM_EXPERTS = 64
HIDDEN = 2048
BLOCK = 1024
_IDX_MASK = NUM_EXPERTS - 1


def _router_body(hs_ref, wt_ref, probs_ref, scores_ref, idx_ref):
    x = hs_ref[...]
    logits = jax.lax.dot_general(
        x, wt_ref[...], (((1,), (0,)), ((), ())),
        preferred_element_type=jnp.float32,
    )
    m = jnp.max(logits, axis=-1, keepdims=True)
    e = jnp.exp(logits - m)
    s = jnp.sum(e, axis=-1, keepdims=True)
    p = e / s
    probs_ref[...] = p

    bits = jax.lax.bitcast_convert_type(p, jnp.int32)
    rev_iota = _IDX_MASK - jax.lax.broadcasted_iota(jnp.int32, p.shape, 1)
    key = (bits & ~_IDX_MASK) | rev_iota
    picks = []
    for _ in range(TOP_K):
        mk = jnp.max(key, axis=-1, keepdims=True)
        picks.append(mk)
        key = jnp.where(key == mk, jnp.iinfo(jnp.int32).min, key)
    k8 = jnp.concatenate(picks, axis=-1)
    idx = _IDX_MASK - (k8 & _IDX_MASK)
    v = jax.lax.bitcast_convert_type(k8 & ~_IDX_MASK, jnp.float32)
    scores_ref[...] = v / jnp.sum(v, axis=-1, keepdims=True)
    idx_ref[...] = idx


@jax.jit
def kernel(hidden_states, W):
    hs = hidden_states.reshape(-1, HIDDEN)
    n = hs.shape[0]
    wt = W.T
    grid = (n // BLOCK,)
    probs, scores, idx = pl.pallas_call(
        _router_body,
        grid=grid,
        in_specs=[
            pl.BlockSpec((BLOCK, HIDDEN), lambda i: (i, 0)),
            pl.BlockSpec((HIDDEN, NUM_EXPERTS), lambda i: (0, 0)),
        ],
        out_specs=[
            pl.BlockSpec((BLOCK, NUM_EXPERTS), lambda i: (i, 0)),
            pl.BlockSpec((BLOCK, TOP_K), lambda i: (i, 0)),
            pl.BlockSpec((BLOCK, TOP_K), lambda i: (i, 0)),
        ],
        out_shape=[
            jax.ShapeDtypeStruct((n, NUM_EXPERTS), jnp.float32),
            jax.ShapeDtypeStruct((n, TOP_K), jnp.float32),
            jax.ShapeDtypeStruct((n, TOP_K), jnp.int32),
        ],
    )(hs, wt)
    return (probs, scores, idx)


# speedup vs baseline: 1.3246x; 1.0232x over previous
import jax
import jax.numpy as jnp
from jax.experimental import pallas as pl

TOP_K = 8
NUM_EXPERTS = 64
HIDDEN = 2048
BLOCK = 2048
_IDX_MASK = NUM_EXPERTS - 1


def _router_body(hs_ref, wt_ref, probs_ref, scores_ref, idx_ref):
    x = hs_ref[...]
    logits = jax.lax.dot_general(
        x, wt_ref[...], (((1,), (0,)), ((), ())),
        preferred_element_type=jnp.float32,
    )
    m = jnp.max(logits, axis=-1, keepdims=True)
    e = jnp.exp(logits - m)
    s = jnp.sum(e, axis=-1, keepdims=True)
    p = e / s
    probs_ref[...] = p

    bits = jax.lax.bitcast_convert_type(p, jnp.int32)
    rev_iota = _IDX_MASK - jax.lax.broadcasted_iota(jnp.int32, p.shape, 1)
    key = (bits & ~_IDX_MASK) | rev_iota
    picks = []
    for _ in range(TOP_K):
        mk = jnp.max(key, axis=-1, keepdims=True)
        picks.append(mk)
        key = jnp.where(key == mk, jnp.iinfo(jnp.int32).min, key)
    k8 = jnp.concatenate(picks, axis=-1)
    idx = _IDX_MASK - (k8 & _IDX_MASK)
    v = jax.lax.bitcast_convert_type(k8 & ~_IDX_MASK, jnp.float32)
    scores_ref[...] = v / jnp.sum(v, axis=-1, keepdims=True)
    idx_ref[...] = idx


@jax.jit
def kernel(hidden_states, W):
    hs = hidden_states.reshape(-1, HIDDEN)
    n = hs.shape[0]
    wt = W.T
    grid = (n // BLOCK,)
    probs, scores, idx = pl.pallas_call(
        _router_body,
        grid=grid,
        in_specs=[
            pl.BlockSpec((BLOCK, HIDDEN), lambda i: (i, 0)),
            pl.BlockSpec((HIDDEN, NUM_EXPERTS), lambda i: (0, 0)),
        ],
        out_specs=[
            pl.BlockSpec((BLOCK, NUM_EXPERTS), lambda i: (i, 0)),
            pl.BlockSpec((BLOCK, TOP_K), lambda i: (i, 0)),
            pl.BlockSpec((BLOCK, TOP_K), lambda i: (i, 0)),
        ],
        out_shape=[
            jax.ShapeDtypeStruct((n, NUM_EXPERTS), jnp.float32),
            jax.ShapeDtypeStruct((n, TOP_K), jnp.float32),
            jax.ShapeDtypeStruct((n, TOP_K), jnp.int32),
        ],
    )(hs, wt)
    return (probs, scores, idx)


# speedup vs baseline: 1.6008x; 1.2085x over previous
import jax
import jax.numpy as jnp
from jax.experimental import pallas as pl

TOP_K = 8
NUM_EXPERTS = 64
HIDDEN = 2048
BLOCK = 2048
CHUNK = 256
_IDX_MASK = NUM_EXPERTS - 1


def _router_body(hs_ref, wt_ref, probs_ref, scores_ref, idx_ref):
    wt = wt_ref[...]
    for c in range(BLOCK // CHUNK):
        rows = pl.ds(c * CHUNK, CHUNK)
        x = hs_ref[rows, :]
        logits = jax.lax.dot_general(
            x, wt, (((1,), (0,)), ((), ())),
            preferred_element_type=jnp.float32,
        )
        m = jnp.max(logits, axis=-1, keepdims=True)
        e = jnp.exp(logits - m)
        s = jnp.sum(e, axis=-1, keepdims=True)
        pn = e / s
        probs_ref[rows, :] = pn

        p = pn.T
        bits = jax.lax.bitcast_convert_type(p, jnp.int32)
        rev_iota = _IDX_MASK - jax.lax.broadcasted_iota(jnp.int32, p.shape, 0)
        key = (bits & ~_IDX_MASK) | rev_iota
        picks = []
        for _ in range(TOP_K):
            mk = jnp.max(key, axis=0, keepdims=True)
            picks.append(mk)
            key = jnp.where(key == mk, jnp.iinfo(jnp.int32).min, key)
        k8 = jnp.concatenate(picks, axis=0)
        idx = _IDX_MASK - (k8 & _IDX_MASK)
        v = jax.lax.bitcast_convert_type(k8 & ~_IDX_MASK, jnp.float32)
        sc = v / jnp.sum(v, axis=0, keepdims=True)
        scores_ref[rows, :] = sc.T
        idx_ref[rows, :] = idx.T


@jax.jit
def kernel(hidden_states, W):
    hs = hidden_states.reshape(-1, HIDDEN)
    n = hs.shape[0]
    wt = W.T
    grid = (n // BLOCK,)
    probs, scores, idx = pl.pallas_call(
        _router_body,
        grid=grid,
        in_specs=[
            pl.BlockSpec((BLOCK, HIDDEN), lambda i: (i, 0)),
            pl.BlockSpec((HIDDEN, NUM_EXPERTS), lambda i: (0, 0)),
        ],
        out_specs=[
            pl.BlockSpec((BLOCK, NUM_EXPERTS), lambda i: (i, 0)),
            pl.BlockSpec((BLOCK, TOP_K), lambda i: (i, 0)),
            pl.BlockSpec((BLOCK, TOP_K), lambda i: (i, 0)),
        ],
        out_shape=[
            jax.ShapeDtypeStruct((n, NUM_EXPERTS), jnp.float32),
            jax.ShapeDtypeStruct((n, TOP_K), jnp.float32),
            jax.ShapeDtypeStruct((n, TOP_K), jnp.int32),
        ],
    )(hs, wt)
    return (probs, scores, idx)
